# baseline (speedup 1.0000x reference)
import functools

import jax
import jax.numpy as jnp
from jax.experimental import pallas as pl
from jax.experimental.pallas import tpu as pltpu

EMBED_DIM = 1024
NUM_EXPERT = 16
TOP_K = 2
EXPERT_DIM = 256
HEAD_DIM = 64
NUM_HEADS = EXPERT_DIM // HEAD_DIM
MAX_POS = 64
SCALING = HEAD_DIM ** (-0.25)


def _gate_kernel(x_ref, wg_ref, idx_ref, gate_ref):
    logits = x_ref[...] @ wg_ref[...]
    e_iota = jax.lax.broadcasted_iota(jnp.int32, logits.shape, 1)
    m1 = jnp.max(logits, axis=1, keepdims=True)
    i1 = jnp.min(jnp.where(logits == m1, e_iota, NUM_EXPERT), axis=1,
                 keepdims=True)
    masked = jnp.where(e_iota == i1, -jnp.inf, logits)
    m2 = jnp.max(masked, axis=1, keepdims=True)
    i2 = jnp.min(jnp.where(masked == m2, e_iota, NUM_EXPERT), axis=1,
                 keepdims=True)
    g1 = jax.nn.sigmoid(m1 - m2)
    idx_ref[...] = jnp.concatenate([i1, i2], axis=1)
    gate_ref[...] = jnp.concatenate([g1, 1.0 - g1], axis=1)


def _qproj_kernel(x_ref, wq_ref, idx_ref, q_ref):
    e = pl.program_id(1)

    @pl.when(e == 0)
    def _():
        q_ref[...] = jnp.zeros_like(q_ref)

    p = ((x_ref[...] @ wq_ref[0]) * SCALING).astype(jnp.bfloat16)
    idx = idx_ref[...]
    for k in range(TOP_K):
        sel = idx[:, k:k + 1] == e
        q_ref[:, k * EXPERT_DIM:(k + 1) * EXPERT_DIM] += jnp.where(
            sel, p, jnp.bfloat16(0))


def _kv_kernel(xk_ref, xv_ref, wk_ref, wv_ref, k_ref, v_ref):
    k_ref[...] = ((xk_ref[...] @ wk_ref[...]) * SCALING).astype(jnp.bfloat16)
    v_ref[...] = (xv_ref[...] @ wv_ref[...]).astype(jnp.bfloat16)


_BC = 512
_BW = 1024


def _attn_kernel(q_ref, k_ref, v_ref, e_ref, y_ref, p_scr, d_scr, *, block_t):
    S = k_ref.shape[0]
    t = pl.program_id(1)
    i0 = t * block_t
    jw = pl.multiple_of(jnp.clip((i0 - 128) // _BC * _BC, 0, S - _BW), _BC)
    rr = jax.lax.broadcasted_iota(jnp.int32, (block_t, _BW), 0)
    cc = jax.lax.broadcasted_iota(jnp.int32, (block_t, _BW), 1)
    d_scr[...] = jnp.clip((jw + cc) - (i0 + rr),
                          1 - MAX_POS, MAX_POS - 1) + MAX_POS
    kk = k_ref[...]
    vv = v_ref[...]
    f32 = jnp.float32
    for h in range(NUM_HEADS):
        qh = q_ref[:, h * HEAD_DIM:(h + 1) * HEAD_DIM]
        kh = kk[:, h * HEAD_DIM:(h + 1) * HEAD_DIM]
        vh = vv[:, h * HEAD_DIM:(h + 1) * HEAD_DIM]
        rlog = jax.lax.dot_general(
            qh, e_ref[h], (((1,), (0,)), ((), ())),
            preferred_element_type=f32)
        rl1 = rlog[:, 1:2]
        rl127 = rlog[:, 127:128]
        rmax = jnp.max(rlog[:, 1:], axis=1, keepdims=True)
        qh32 = qh.astype(f32)
        qn2 = jnp.sum(qh32 * qh32, axis=1, keepdims=True)
        kh32 = kh.astype(f32)
        kn2 = jnp.sum(kh32 * kh32, axis=1, keepdims=True)
        m2 = jnp.sqrt(qn2 * jnp.max(kn2)) + rmax - rl1
        l = None
        for c in range(S // _BC):
            ks = kh[c * _BC:(c + 1) * _BC, :]
            s = jax.lax.dot_general(
                qh, ks, (((1,), (1,)), ((), ())),
                preferred_element_type=f32)
            base = jnp.where(c * _BC >= jw + _BW, rl127 - rl1, 0.0)
            p = jnp.exp(s + (base - m2))
            lc = jnp.sum(p, axis=1, keepdims=True)
            l = lc if l is None else l + lc
            p_scr[:, c * _BC:(c + 1) * _BC] = p.astype(jnp.bfloat16)
        delta = jnp.take_along_axis(rlog, d_scr[...], axis=1) - rl1
        pold = p_scr[:, pl.ds(jw, _BW)].astype(f32)
        pnew = pold * jnp.exp(delta)
        l = l + jnp.sum(pnew - pold, axis=1, keepdims=True)
        p_scr[:, pl.ds(jw, _BW)] = pnew.astype(jnp.bfloat16)
        pv = jax.lax.dot_general(
            p_scr[...], vh, (((1,), (0,)), ((), ())),
            preferred_element_type=f32)
        y_ref[:, h * HEAD_DIM:(h + 1) * HEAD_DIM] = pv / l


def _oproj_kernel(y_ref, idx_ref, gate_ref, wo_ref, o_ref):
    e = pl.program_id(1)

    @pl.when(e == 0)
    def _():
        o_ref[...] = jnp.zeros_like(o_ref)

    idx = idx_ref[...]
    g = gate_ref[...]
    z = None
    for k in range(TOP_K):
        w = jnp.where(idx[:, k:k + 1] == e, g[:, k:k + 1], 0.0)
        zk = y_ref[:, k * EXPERT_DIM:(k + 1) * EXPERT_DIM] * w
        z = zk if z is None else z + zk
    o_ref[...] += z @ wo_ref[0]


def kernel(query, key, value, Wg, Wq, Wk, Wv, Wo, rel_pos_emb):
    T, B, D = query.shape
    S = key.shape[0]
    n = T * B
    x = query.reshape(n, D)
    xk = key.reshape(S * B, D)
    xv = value.reshape(S * B, D)
    f32 = jnp.float32
    bf16 = jnp.bfloat16

    bT = 512
    idx, gates = pl.pallas_call(
        _gate_kernel,
        grid=(n // bT,),
        in_specs=[
            pl.BlockSpec((bT, D), lambda i: (i, 0)),
            pl.BlockSpec((D, NUM_EXPERT), lambda i: (0, 0)),
        ],
        out_specs=[
            pl.BlockSpec((bT, TOP_K), lambda i: (i, 0)),
            pl.BlockSpec((bT, TOP_K), lambda i: (i, 0)),
        ],
        out_shape=[
            jax.ShapeDtypeStruct((n, TOP_K), jnp.int32),
            jax.ShapeDtypeStruct((n, TOP_K), f32),
        ],
    )(x, Wg)

    q = pl.pallas_call(
        _qproj_kernel,
        grid=(n // bT, NUM_EXPERT),
        in_specs=[
            pl.BlockSpec((bT, D), lambda i, e: (i, 0)),
            pl.BlockSpec((1, D, EXPERT_DIM), lambda i, e: (e, 0, 0)),
            pl.BlockSpec((bT, TOP_K), lambda i, e: (i, 0)),
        ],
        out_specs=pl.BlockSpec((bT, TOP_K * EXPERT_DIM), lambda i, e: (i, 0)),
        out_shape=jax.ShapeDtypeStruct((n, TOP_K * EXPERT_DIM), bf16),
    )(x, Wq, idx)

    kp, vp = pl.pallas_call(
        _kv_kernel,
        grid=(S * B // bT,),
        in_specs=[
            pl.BlockSpec((bT, D), lambda i: (i, 0)),
            pl.BlockSpec((bT, D), lambda i: (i, 0)),
            pl.BlockSpec((D, EXPERT_DIM), lambda i: (0, 0)),
            pl.BlockSpec((D, EXPERT_DIM), lambda i: (0, 0)),
        ],
        out_specs=[
            pl.BlockSpec((bT, EXPERT_DIM), lambda i: (i, 0)),
            pl.BlockSpec((bT, EXPERT_DIM), lambda i: (i, 0)),
        ],
        out_shape=[
            jax.ShapeDtypeStruct((S * B, EXPERT_DIM), bf16),
            jax.ShapeDtypeStruct((S * B, EXPERT_DIM), bf16),
        ],
    )(xk, xv, Wk, Wv)

    rpe = rel_pos_emb[:, :, :2 * MAX_POS].astype(bf16)

    bA = 256
    y = pl.pallas_call(
        functools.partial(_attn_kernel, block_t=bA),
        grid=(TOP_K, T // bA),
        in_specs=[
            pl.BlockSpec((bA, EXPERT_DIM), lambda k, t: (t, k)),
            pl.BlockSpec((S, EXPERT_DIM), lambda k, t: (0, 0)),
            pl.BlockSpec((S, EXPERT_DIM), lambda k, t: (0, 0)),
            pl.BlockSpec((NUM_HEADS, HEAD_DIM, 2 * MAX_POS),
                         lambda k, t: (0, 0, 0)),
        ],
        out_specs=pl.BlockSpec((bA, EXPERT_DIM), lambda k, t: (t, k)),
        out_shape=jax.ShapeDtypeStruct((n, TOP_K * EXPERT_DIM), f32),
        scratch_shapes=[
            pltpu.VMEM((bA, S), jnp.bfloat16),
            pltpu.VMEM((bA, _BW), jnp.int32),
        ],
    )(q, kp, vp, rpe)

    out = pl.pallas_call(
        _oproj_kernel,
        grid=(n // bT, NUM_EXPERT),
        in_specs=[
            pl.BlockSpec((bT, TOP_K * EXPERT_DIM), lambda i, e: (i, 0)),
            pl.BlockSpec((bT, TOP_K), lambda i, e: (i, 0)),
            pl.BlockSpec((bT, TOP_K), lambda i, e: (i, 0)),
            pl.BlockSpec((1, EXPERT_DIM, D), lambda i, e: (e, 0, 0)),
        ],
        out_specs=pl.BlockSpec((bT, D), lambda i, e: (i, 0)),
        out_shape=jax.ShapeDtypeStruct((n, D), f32),
    )(y, idx, gates, Wo)

    return out.reshape(T, B, D)
